# baseline (speedup 1.0000x reference)
import numpy as np
import jax
import jax.numpy as jnp
from jax.experimental import pallas as pl
from jax.experimental.pallas import tpu as pltpu

N_BINS = 15
_BOUNDS = np.linspace(0.0, 1.0, N_BINS + 1).astype(np.float32)


def _ece_body(x_ref, lab_ref, out_ref, acc_ref):
    i = pl.program_id(0)
    nsteps = pl.num_programs(0)

    @pl.when(i == 0)
    def _init():
        acc_ref[...] = jnp.zeros_like(acc_ref)

    x = x_ref[...]
    b = x.shape[0]
    conf = jnp.max(x, axis=1, keepdims=True)
    col = jax.lax.broadcasted_iota(jnp.int32, x.shape, 1)
    pred = jnp.min(jnp.where(x == conf, col, jnp.int32(1 << 30)),
                   axis=1, keepdims=True)
    lab = lab_ref[0]
    accv = (pred == lab).astype(jnp.float32)

    binidx = jnp.zeros((b, 1), jnp.int32)
    for k in range(1, N_BINS):
        binidx += (conf > _BOUNDS[k]).astype(jnp.int32)

    lane = jax.lax.broadcasted_iota(jnp.int32, (b, 128), 1)
    onehot = (binidx == lane).astype(jnp.float32)
    acc_ref[0:1, :] += jnp.sum(onehot, axis=0, keepdims=True)
    acc_ref[1:2, :] += jnp.sum(onehot * conf, axis=0, keepdims=True)
    acc_ref[2:3, :] += jnp.sum(onehot * accv, axis=0, keepdims=True)

    @pl.when(i == nsteps - 1)
    def _fini():
        n_total = jnp.float32(b) * jnp.float32(nsteps)
        c = acc_ref[0:1, :]
        sc = acc_ref[1:2, :]
        sa = acc_ref[2:3, :]
        lane_id = jax.lax.broadcasted_iota(jnp.int32, (1, 128), 1)
        safe = jnp.maximum(c, 1.0)
        gap = jnp.abs(sc / safe - sa / safe) * (c / n_total)
        gap = jnp.where((c > 0.0) & (lane_id < N_BINS), gap, 0.0)
        out_ref[...] = jnp.sum(gap).reshape(1, 1)


def kernel(softmaxes, labels):
    n, c = softmaxes.shape
    blk = 2000 if n % 2000 == 0 else 8
    nb = n // blk
    lab3 = labels.astype(jnp.int32).reshape(nb, blk, 1)
    out = pl.pallas_call(
        _ece_body,
        grid=(nb,),
        in_specs=[
            pl.BlockSpec((blk, c), lambda i: (i, 0)),
            pl.BlockSpec((1, blk, 1), lambda i: (i, 0, 0)),
        ],
        out_specs=pl.BlockSpec((1, 1), lambda i: (0, 0)),
        out_shape=jax.ShapeDtypeStruct((1, 1), jnp.float32),
        scratch_shapes=[pltpu.VMEM((8, 128), jnp.float32)],
    )(softmaxes, lab3)
    return out.reshape(1)
